# baseline (speedup 1.0000x reference)
import functools

import jax
import jax.numpy as jnp
from jax import lax
from jax.experimental import pallas as pl
from jax.experimental.pallas import tpu as pltpu
from jax.experimental.pallas import tpu_sc as plsc

N = 207
NP = 208
HID = 8
B = 8
S = 12
T = B * S
TP = 128
E = 3312
EP = 3328
NW = 32
EPW = EP // NW
H = N * HID
G4 = 4 * H


def _gat_edges_sc(xT, src, dst, csv, cdv, zrows):
    mesh = plsc.VectorSubcoreMesh(core_axis_name="c", subcore_axis_name="s")

    @functools.partial(
        pl.kernel,
        out_type=(
            jax.ShapeDtypeStruct((2, NP, TP), jnp.float32),
            jax.ShapeDtypeStruct((2, NP, TP), jnp.float32),
        ),
        mesh=mesh,
        scratch_types=[
            pltpu.VMEM((EPW,), jnp.int32),
            pltpu.VMEM((EPW,), jnp.int32),
            pltpu.VMEM((EPW, TP), jnp.float32),
            pltpu.VMEM((EPW, TP), jnp.float32),
            pltpu.VMEM((EPW, TP), jnp.float32),
            pltpu.VMEM((EPW, TP), jnp.float32),
            pltpu.VMEM((16,), jnp.float32),
            pltpu.VMEM((16,), jnp.float32),
            pltpu.VMEM_SHARED((NP, TP), jnp.float32),
            pltpu.VMEM_SHARED((NP, TP), jnp.float32),
            pltpu.SemaphoreType.DMA,
        ],
    )
    def k(xT_hbm, src_hbm, dst_hbm, cs_hbm, cd_hbm, z_hbm,
          num_hbm, den_hbm,
          src_v, dst_v, xs_v, xd_v, e_v, n_v, cs_v, cd_v,
          num_sh, den_sh, sem):
        c = lax.axis_index("c")
        sid = lax.axis_index("s")
        base = (c * 16 + sid) * EPW
        pltpu.sync_copy(src_hbm.at[pl.ds(base, EPW)], src_v)
        pltpu.sync_copy(dst_hbm.at[pl.ds(base, EPW)], dst_v)
        pltpu.sync_copy(cs_hbm, cs_v)
        pltpu.sync_copy(cd_hbm, cd_v)

        @pl.when(sid == 0)
        def _():
            pltpu.sync_copy(z_hbm, num_sh)
            pltpu.sync_copy(z_hbm, den_sh)

        pltpu.async_copy(xT_hbm.at[src_v], xs_v, sem).wait()
        pltpu.async_copy(xT_hbm.at[dst_v], xd_v, sem).wait()
        cs = cs_v[...]
        cd = cd_v[...]

        @pl.loop(0, EPW)
        def _(r):
            for j in range(TP // 16):
                sl = pl.ds(j * 16, 16)
                xs = xs_v[r, sl]
                xd = xd_v[r, sl]
                a = xs * cs + xd * cd
                a = jnp.where(a > 0.0, a, 0.2 * a)
                e = jnp.exp(a)
                e_v[r, sl] = e
                n_v[r, sl] = e * xs

        plsc.subcore_barrier()
        pltpu.sync_copy(n_v, num_sh.at[dst_v], add=True)
        pltpu.sync_copy(e_v, den_sh.at[dst_v], add=True)
        plsc.subcore_barrier()

        @pl.when(sid == 0)
        def _():
            pltpu.sync_copy(num_sh, num_hbm.at[c])
            pltpu.sync_copy(den_sh, den_hbm.at[c])

    return k(xT, src, dst, csv, cdv, zrows)


def _dotT(a, b):
    return lax.dot_general(a, b, (((1,), (1,)), ((), ())),
                           preferred_element_type=jnp.float32)


def _dotTb(a, b):
    return lax.dot_general(a.astype(jnp.bfloat16), b, (((1,), (1,)), ((), ())),
                           preferred_element_type=jnp.float32)


def _prep_u0_body(num_ref, den_ref, wgt_ref, bg_ref, wih_ref, bias_ref, u_ref):
    s_mat = (num_ref[0] + num_ref[1]) / (den_ref[0] + den_ref[1] + 1e-16)
    ii = lax.broadcasted_iota(jnp.int32, (T, TP), 0)
    jj = lax.broadcasted_iota(jnp.int32, (T, TP), 1)
    eye = jnp.where(ii == jj, 1.0, 0.0)
    sT = _dotT(eye, s_mat)
    x = jnp.maximum(_dotT(sT, wgt_ref[...]) + bg_ref[...][None, :], 0.0)
    for k in range(4):
        u_ref[k] = _dotTb(x, wih_ref[k]) + bias_ref[k][None, :]


def _u1_body(h1_ref, wih_ref, bias_ref, u_ref):
    h1 = h1_ref[...]
    for k in range(4):
        u_ref[k] = _dotTb(h1, wih_ref[k]) + bias_ref[k][None, :]


def _scan_body(u_ref, whh_ref, h_out_ref):
    def step(s, carry):
        h, c = carry
        row = pl.multiple_of(8 * s, 8)
        gi = u_ref[0, pl.ds(row, B), :] + _dotTb(h, whh_ref[0])
        gf = u_ref[1, pl.ds(row, B), :] + _dotTb(h, whh_ref[1])
        gg = u_ref[2, pl.ds(row, B), :] + _dotTb(h, whh_ref[2])
        go = u_ref[3, pl.ds(row, B), :] + _dotTb(h, whh_ref[3])
        c = jax.nn.sigmoid(gf) * c + jax.nn.sigmoid(gi) * jnp.tanh(gg)
        h = jax.nn.sigmoid(go) * jnp.tanh(c)
        h_out_ref[pl.ds(row, B), :] = h
        return h, c

    z = jnp.zeros((B, H), jnp.float32)
    lax.fori_loop(0, S, step, (z, z))


def _scan_head_body(u_ref, whh_ref, wl_ref, bl_ref, out_ref):
    def step(s, carry):
        h, c = carry
        row = pl.multiple_of(8 * s, 8)
        gi = u_ref[0, pl.ds(row, B), :] + _dotTb(h, whh_ref[0])
        gf = u_ref[1, pl.ds(row, B), :] + _dotTb(h, whh_ref[1])
        gg = u_ref[2, pl.ds(row, B), :] + _dotTb(h, whh_ref[2])
        go = u_ref[3, pl.ds(row, B), :] + _dotTb(h, whh_ref[3])
        c = jax.nn.sigmoid(gf) * c + jax.nn.sigmoid(gi) * jnp.tanh(gg)
        h = jax.nn.sigmoid(go) * jnp.tanh(c)
        return h, c

    z = jnp.zeros((B, H), jnp.float32)
    h, _ = lax.fori_loop(0, S, step, (z, z))
    out_ref[...] = _dotT(h, wl_ref[...]) + bl_ref[...][None, :]


def _tc_call(body, out_shape, *args):
    return pl.pallas_call(
        body,
        out_shape=jax.ShapeDtypeStruct(out_shape, jnp.float32),
    )(*args)


def kernel(x_sequence, edge_index, W_gat, a_src, a_dst, b_gat,
           Wih0, Whh0, bih0, bhh0, Wih1, Whh1, bih1, bhh1, W_lin, b_lin):
    f32 = jnp.float32
    xT = jnp.transpose(x_sequence, (2, 1, 0)).reshape(N, T)
    xT = jnp.pad(xT, ((0, 1), (0, TP - T)))
    pad = jnp.full((EP - E,), N, jnp.int32)
    src = jnp.concatenate([edge_index[0], pad])
    dst = jnp.concatenate([edge_index[1], pad])
    wrow = W_gat[0]
    csv = jnp.full((16,), jnp.dot(wrow, a_src), f32)
    cdv = jnp.full((16,), jnp.dot(wrow, a_dst), f32)
    zrows = jnp.zeros((NP, TP), f32)
    wg_big = jnp.kron(jnp.eye(N, NP, dtype=f32), wrow[:, None])
    bg_big = jnp.tile(b_gat, N)
    bf16 = jnp.bfloat16
    wih0 = Wih0.reshape(4, H, H).astype(bf16)
    whh0 = Whh0.reshape(4, H, H).astype(bf16)
    wih1 = Wih1.reshape(4, H, H).astype(bf16)
    whh1 = Whh1.reshape(4, H, H).astype(bf16)
    bias0 = (bih0 + bhh0).reshape(4, H)
    bias1 = (bih1 + bhh1).reshape(4, H)

    num_p, den_p = _gat_edges_sc(xT, src, dst, csv, cdv, zrows)

    u0 = _tc_call(_prep_u0_body, (4, T, H),
                  num_p, den_p, wg_big, bg_big, wih0, bias0)
    h1 = _tc_call(_scan_body, (T, H), u0, whh0)
    u1 = _tc_call(_u1_body, (4, T, H), h1, wih1, bias1)
    out = _tc_call(_scan_head_body, (B, N), u1, whh1, W_lin, b_lin)
    return out


# speedup vs baseline: 1.1883x; 1.1883x over previous
import functools

import jax
import jax.numpy as jnp
from jax import lax
from jax.experimental import pallas as pl
from jax.experimental.pallas import tpu as pltpu
from jax.experimental.pallas import tpu_sc as plsc

N = 207
NP = 208
HID = 8
B = 8
S = 12
T = B * S
TP = 128
E = 3312
EP = 3328
NW = 32
EPW = EP // NW
H = N * HID
CH = 72


def _gat_edges_sc(xT, src, dst, csv, cdv, zrows):
    mesh = plsc.VectorSubcoreMesh(core_axis_name="c", subcore_axis_name="s")

    @functools.partial(
        pl.kernel,
        out_type=(
            jax.ShapeDtypeStruct((2, NP, TP), jnp.float32),
            jax.ShapeDtypeStruct((2, NP, TP), jnp.float32),
        ),
        mesh=mesh,
        scratch_types=[
            pltpu.VMEM((EPW,), jnp.int32),
            pltpu.VMEM((EPW,), jnp.int32),
            pltpu.VMEM((EPW, TP), jnp.float32),
            pltpu.VMEM((EPW, TP), jnp.float32),
            pltpu.VMEM((EPW, TP), jnp.float32),
            pltpu.VMEM((EPW, TP), jnp.float32),
            pltpu.VMEM((16,), jnp.float32),
            pltpu.VMEM((16,), jnp.float32),
            pltpu.VMEM_SHARED((NP, TP), jnp.float32),
            pltpu.VMEM_SHARED((NP, TP), jnp.float32),
            pltpu.SemaphoreType.DMA,
        ],
    )
    def k(xT_hbm, src_hbm, dst_hbm, cs_hbm, cd_hbm, z_hbm,
          num_hbm, den_hbm,
          src_v, dst_v, xs_v, xd_v, e_v, n_v, cs_v, cd_v,
          num_sh, den_sh, sem):
        c = lax.axis_index("c")
        sid = lax.axis_index("s")
        base = (c * 16 + sid) * EPW
        pltpu.sync_copy(src_hbm.at[pl.ds(base, EPW)], src_v)
        pltpu.sync_copy(dst_hbm.at[pl.ds(base, EPW)], dst_v)
        pltpu.sync_copy(cs_hbm, cs_v)
        pltpu.sync_copy(cd_hbm, cd_v)

        @pl.when(sid == 0)
        def _():
            pltpu.sync_copy(z_hbm, num_sh)
            pltpu.sync_copy(z_hbm, den_sh)

        pltpu.async_copy(xT_hbm.at[src_v], xs_v, sem).wait()
        pltpu.async_copy(xT_hbm.at[dst_v], xd_v, sem).wait()
        cs = cs_v[...]
        cd = cd_v[...]

        @pl.loop(0, EPW)
        def _(r):
            for j in range(TP // 16):
                sl = pl.ds(j * 16, 16)
                xs = xs_v[r, sl]
                xd = xd_v[r, sl]
                a = xs * cs + xd * cd
                a = jnp.where(a > 0.0, a, 0.2 * a)
                e = jnp.exp(a)
                e_v[r, sl] = e
                n_v[r, sl] = e * xs

        plsc.subcore_barrier()
        pltpu.sync_copy(n_v, num_sh.at[dst_v], add=True)
        pltpu.sync_copy(e_v, den_sh.at[dst_v], add=True)
        plsc.subcore_barrier()

        @pl.when(sid == 0)
        def _():
            pltpu.sync_copy(num_sh, num_hbm.at[c])
            pltpu.sync_copy(den_sh, den_hbm.at[c])

    return k(xT, src, dst, csv, cdv, zrows)


def _dotT(a, b):
    return lax.dot_general(a, b, (((1,), (1,)), ((), ())),
                           preferred_element_type=jnp.float32)


def _dotTb(a, b):
    return lax.dot_general(a.astype(jnp.bfloat16), b, (((1,), (1,)), ((), ())),
                           preferred_element_type=jnp.float32)


def _dense_body(num_ref, den_ref, wgt_ref, bg_ref, bias0_ref, bias1_ref,
                wl_ref, bl_ref, wih0_hbm, whh0_hbm, wih1_hbm, whh1_hbm,
                out_ref,
                fwin, wbf, u0, u1, h1, sem):
    def fetch(w_hbm, k):
        cp = pltpu.make_async_copy(w_hbm.at[k], fwin, sem)
        cp.start()
        cp.wait()

    def load_bf16(w_hbm):
        for k in range(4):
            fetch(w_hbm, k)

            @pl.loop(0, H // CH)
            def _(i):
                r = pl.ds(pl.multiple_of(i * CH, 8), CH)
                wbf[k, r, :] = fwin[r, :].astype(jnp.bfloat16)

    s_mat = (num_ref[0] + num_ref[1]) / (den_ref[0] + den_ref[1] + 1e-16)
    ii = lax.broadcasted_iota(jnp.int32, (T, TP), 0)
    jj = lax.broadcasted_iota(jnp.int32, (T, TP), 1)
    eye = jnp.where(ii == jj, 1.0, 0.0)
    sT = _dotT(eye, s_mat)
    x = jnp.maximum(_dotT(sT, wgt_ref[...]) + bg_ref[...][None, :], 0.0)

    for k in range(4):
        fetch(wih0_hbm, k)
        u0[k] = _dotT(x, fwin[...]) + bias0_ref[k][None, :]

    load_bf16(whh0_hbm)

    def step0(s, carry):
        h, c = carry
        row = pl.multiple_of(8 * s, 8)
        gi = u0[0, pl.ds(row, B), :] + _dotTb(h, wbf[0])
        gf = u0[1, pl.ds(row, B), :] + _dotTb(h, wbf[1])
        gg = u0[2, pl.ds(row, B), :] + _dotTb(h, wbf[2])
        go = u0[3, pl.ds(row, B), :] + _dotTb(h, wbf[3])
        c = jax.nn.sigmoid(gf) * c + jax.nn.sigmoid(gi) * jnp.tanh(gg)
        h = jax.nn.sigmoid(go) * jnp.tanh(c)
        h1[pl.ds(row, B), :] = h
        return h, c

    z = jnp.zeros((B, H), jnp.float32)
    lax.fori_loop(0, S, step0, (z, z))

    h1v = h1[...]
    for k in range(4):
        fetch(wih1_hbm, k)
        u1[k] = _dotT(h1v, fwin[...]) + bias1_ref[k][None, :]

    load_bf16(whh1_hbm)

    def step1(s, carry):
        h, c = carry
        row = pl.multiple_of(8 * s, 8)
        gi = u1[0, pl.ds(row, B), :] + _dotTb(h, wbf[0])
        gf = u1[1, pl.ds(row, B), :] + _dotTb(h, wbf[1])
        gg = u1[2, pl.ds(row, B), :] + _dotTb(h, wbf[2])
        go = u1[3, pl.ds(row, B), :] + _dotTb(h, wbf[3])
        c = jax.nn.sigmoid(gf) * c + jax.nn.sigmoid(gi) * jnp.tanh(gg)
        h = jax.nn.sigmoid(go) * jnp.tanh(c)
        return h, c

    h, _ = lax.fori_loop(0, S, step1, (z, z))
    out_ref[...] = _dotT(h, wl_ref[...]) + bl_ref[...][None, :]


def _dense_chain(num_p, den_p, wg_big, bg_big, bias0, bias1, wlin, blin,
                 wih0, whh0, wih1, whh1):
    vmem = pl.BlockSpec(memory_space=pltpu.VMEM)
    hbm = pl.BlockSpec(memory_space=pl.ANY)
    return pl.pallas_call(
        _dense_body,
        in_specs=[vmem] * 8 + [hbm] * 4,
        out_specs=vmem,
        out_shape=jax.ShapeDtypeStruct((B, N), jnp.float32),
        scratch_shapes=[
            pltpu.VMEM((H, H), jnp.float32),
            pltpu.VMEM((4, H, H), jnp.bfloat16),
            pltpu.VMEM((4, T, H), jnp.float32),
            pltpu.VMEM((4, T, H), jnp.float32),
            pltpu.VMEM((T, H), jnp.float32),
            pltpu.SemaphoreType.DMA,
        ],
    )(num_p, den_p, wg_big, bg_big, bias0, bias1, wlin, blin,
      wih0, whh0, wih1, whh1)


def kernel(x_sequence, edge_index, W_gat, a_src, a_dst, b_gat,
           Wih0, Whh0, bih0, bhh0, Wih1, Whh1, bih1, bhh1, W_lin, b_lin):
    f32 = jnp.float32
    xT = jnp.transpose(x_sequence, (2, 1, 0)).reshape(N, T)
    xT = jnp.pad(xT, ((0, 1), (0, TP - T)))
    pad = jnp.full((EP - E,), N, jnp.int32)
    src = jnp.concatenate([edge_index[0], pad])
    dst = jnp.concatenate([edge_index[1], pad])
    wrow = W_gat[0]
    csv = jnp.full((16,), jnp.dot(wrow, a_src), f32)
    cdv = jnp.full((16,), jnp.dot(wrow, a_dst), f32)
    zrows = jnp.zeros((NP, TP), f32)
    wg_big = jnp.kron(jnp.eye(N, NP, dtype=f32), wrow[:, None])
    bg_big = jnp.tile(b_gat, N)
    wih0 = Wih0.reshape(4, H, H)
    whh0 = Whh0.reshape(4, H, H)
    wih1 = Wih1.reshape(4, H, H)
    whh1 = Whh1.reshape(4, H, H)
    bias0 = (bih0 + bhh0).reshape(4, H)
    bias1 = (bih1 + bhh1).reshape(4, H)

    num_p, den_p = _gat_edges_sc(xT, src, dst, csv, cdv, zrows)

    return _dense_chain(num_p, den_p, wg_big, bg_big, bias0, bias1,
                        W_lin, b_lin, wih0, whh0, wih1, whh1)


# speedup vs baseline: 1.3467x; 1.1333x over previous
import functools

import jax
import jax.numpy as jnp
from jax import lax
from jax.experimental import pallas as pl
from jax.experimental.pallas import tpu as pltpu
from jax.experimental.pallas import tpu_sc as plsc

N = 207
NP = 208
HID = 8
B = 8
S = 12
T = B * S
TP = 128
E = 3312
EP = 3328
NW = 32
EPW = EP // NW
H = N * HID
CH = 72


def _gat_edges_sc(xT, src, dst, csv, cdv, zrows):
    mesh = plsc.VectorSubcoreMesh(core_axis_name="c", subcore_axis_name="s")

    @functools.partial(
        pl.kernel,
        out_type=(
            jax.ShapeDtypeStruct((2, NP, TP), jnp.float32),
            jax.ShapeDtypeStruct((2, NP, TP), jnp.float32),
        ),
        mesh=mesh,
        scratch_types=[
            pltpu.VMEM((EPW,), jnp.int32),
            pltpu.VMEM((EPW,), jnp.int32),
            pltpu.VMEM((EPW, TP), jnp.float32),
            pltpu.VMEM((EPW, TP), jnp.float32),
            pltpu.VMEM((EPW, TP), jnp.float32),
            pltpu.VMEM((EPW, TP), jnp.float32),
            pltpu.VMEM((16,), jnp.float32),
            pltpu.VMEM((16,), jnp.float32),
            pltpu.VMEM_SHARED((NP, TP), jnp.float32),
            pltpu.VMEM_SHARED((NP, TP), jnp.float32),
            pltpu.SemaphoreType.DMA,
        ],
    )
    def k(xT_hbm, src_hbm, dst_hbm, cs_hbm, cd_hbm, z_hbm,
          num_hbm, den_hbm,
          src_v, dst_v, xs_v, xd_v, e_v, n_v, cs_v, cd_v,
          num_sh, den_sh, sem):
        c = lax.axis_index("c")
        sid = lax.axis_index("s")
        base = (c * 16 + sid) * EPW
        pltpu.sync_copy(src_hbm.at[pl.ds(base, EPW)], src_v)
        pltpu.sync_copy(dst_hbm.at[pl.ds(base, EPW)], dst_v)
        pltpu.sync_copy(cs_hbm, cs_v)
        pltpu.sync_copy(cd_hbm, cd_v)

        @pl.when(sid == 0)
        def _():
            pltpu.sync_copy(z_hbm, num_sh)
            pltpu.sync_copy(z_hbm, den_sh)

        pltpu.async_copy(xT_hbm.at[src_v], xs_v, sem).wait()
        pltpu.async_copy(xT_hbm.at[dst_v], xd_v, sem).wait()
        cs = cs_v[...]
        cd = cd_v[...]

        @pl.loop(0, EPW)
        def _(r):
            for j in range(TP // 16):
                sl = pl.ds(j * 16, 16)
                xs = xs_v[r, sl]
                xd = xd_v[r, sl]
                a = xs * cs + xd * cd
                a = jnp.where(a > 0.0, a, 0.2 * a)
                e = jnp.exp(a)
                e_v[r, sl] = e
                n_v[r, sl] = e * xs

        plsc.subcore_barrier()
        pltpu.sync_copy(n_v, num_sh.at[dst_v], add=True)
        pltpu.sync_copy(e_v, den_sh.at[dst_v], add=True)
        plsc.subcore_barrier()

        @pl.when(sid == 0)
        def _():
            pltpu.sync_copy(num_sh, num_hbm.at[c])
            pltpu.sync_copy(den_sh, den_hbm.at[c])

    return k(xT, src, dst, csv, cdv, zrows)


def _dotT(a, b):
    return lax.dot_general(a, b, (((1,), (1,)), ((), ())),
                           preferred_element_type=jnp.float32)


def _dotTb(a, b):
    return lax.dot_general(a.astype(jnp.bfloat16), b, (((1,), (1,)), ((), ())),
                           preferred_element_type=jnp.float32)


def _dense_body(num_ref, den_ref, wgt_ref, bg_ref, bias0_ref, bias1_ref,
                wl_ref, bl_ref, wih0_hbm, whh0_hbm, wih1_hbm, whh1_hbm,
                out_ref,
                fwin, wbf, u0, h1, sem):
    seq = ([(wih0_hbm, k) for k in range(4)] + [(whh0_hbm, k) for k in range(4)]
           + [(wih1_hbm, k) for k in range(4)] + [(whh1_hbm, k) for k in range(4)])

    def _cp(i):
        w_hbm, k = seq[i]
        return pltpu.make_async_copy(w_hbm.at[k], fwin.at[i % 2], sem.at[i % 2])

    def start(i):
        if i < len(seq):
            _cp(i).start()

    def cast_slab(i, k):
        @pl.loop(0, H // CH)
        def _(j):
            r = pl.ds(pl.multiple_of(j * CH, 8), CH)
            wbf[k, r, :] = fwin[i % 2, r, :].astype(jnp.bfloat16)

    def lstm_step(u, h, c, s):
        row = pl.multiple_of(8 * s, 8)
        gi = u[0, pl.ds(row, B), :] + _dotTb(h, wbf[0])
        gf = u[1, pl.ds(row, B), :] + _dotTb(h, wbf[1])
        gg = u[2, pl.ds(row, B), :] + _dotTb(h, wbf[2])
        go = u[3, pl.ds(row, B), :] + _dotTb(h, wbf[3])
        c = jax.nn.sigmoid(gf) * c + jax.nn.sigmoid(gi) * jnp.tanh(gg)
        h = jax.nn.sigmoid(go) * jnp.tanh(c)
        return h, c

    start(0)
    start(1)

    s_mat = (num_ref[0] + num_ref[1]) / (den_ref[0] + den_ref[1] + 1e-16)
    ii = lax.broadcasted_iota(jnp.int32, (T, TP), 0)
    jj = lax.broadcasted_iota(jnp.int32, (T, TP), 1)
    eye = jnp.where(ii == jj, 1.0, 0.0)
    sT = _dotT(eye, s_mat)
    x = jnp.maximum(_dotT(sT, wgt_ref[...]) + bg_ref[...][None, :], 0.0)

    for k in range(4):
        i = k
        _cp(i).wait()
        u0[k] = _dotT(x, fwin[i % 2]) + bias0_ref[k][None, :]
        start(i + 2)

    for k in range(4):
        i = 4 + k
        _cp(i).wait()
        cast_slab(i, k)
        start(i + 2)

    def step0(s, carry):
        h, c = lstm_step(u0, *carry, s)
        h1[pl.ds(pl.multiple_of(8 * s, 8), B), :] = h
        return h, c

    z = jnp.zeros((B, H), jnp.float32)
    lax.fori_loop(0, S, step0, (z, z))

    h1v = h1[...]
    for k in range(4):
        i = 8 + k
        _cp(i).wait()
        u0[k] = _dotT(h1v, fwin[i % 2]) + bias1_ref[k][None, :]
        start(i + 2)

    for k in range(4):
        i = 12 + k
        _cp(i).wait()
        cast_slab(i, k)
        start(i + 2)

    def step1(s, carry):
        return lstm_step(u0, *carry, s)

    h, _ = lax.fori_loop(0, S, step1, (z, z))
    out_ref[...] = _dotT(h, wl_ref[...]) + bl_ref[...][None, :]


def _dense_chain(num_p, den_p, wg_big, bg_big, bias0, bias1, wlin, blin,
                 wih0, whh0, wih1, whh1):
    vmem = pl.BlockSpec(memory_space=pltpu.VMEM)
    hbm = pl.BlockSpec(memory_space=pl.ANY)
    return pl.pallas_call(
        _dense_body,
        in_specs=[vmem] * 8 + [hbm] * 4,
        out_specs=vmem,
        out_shape=jax.ShapeDtypeStruct((B, N), jnp.float32),
        scratch_shapes=[
            pltpu.VMEM((2, H, H), jnp.float32),
            pltpu.VMEM((4, H, H), jnp.bfloat16),
            pltpu.VMEM((4, T, H), jnp.float32),
            pltpu.VMEM((T, H), jnp.float32),
            pltpu.SemaphoreType.DMA((2,)),
        ],
    )(num_p, den_p, wg_big, bg_big, bias0, bias1, wlin, blin,
      wih0, whh0, wih1, whh1)


def kernel(x_sequence, edge_index, W_gat, a_src, a_dst, b_gat,
           Wih0, Whh0, bih0, bhh0, Wih1, Whh1, bih1, bhh1, W_lin, b_lin):
    f32 = jnp.float32
    xT = jnp.transpose(x_sequence, (2, 1, 0)).reshape(N, T)
    xT = jnp.pad(xT, ((0, 1), (0, TP - T)))
    pad = jnp.full((EP - E,), N, jnp.int32)
    src = jnp.concatenate([edge_index[0], pad])
    dst = jnp.concatenate([edge_index[1], pad])
    wrow = W_gat[0]
    csv = jnp.full((16,), jnp.dot(wrow, a_src), f32)
    cdv = jnp.full((16,), jnp.dot(wrow, a_dst), f32)
    zrows = jnp.zeros((NP, TP), f32)
    wg_big = jnp.kron(jnp.eye(N, NP, dtype=f32), wrow[:, None])
    bg_big = jnp.tile(b_gat, N)
    wih0 = Wih0.reshape(4, H, H)
    whh0 = Whh0.reshape(4, H, H)
    wih1 = Wih1.reshape(4, H, H)
    whh1 = Whh1.reshape(4, H, H)
    bias0 = (bih0 + bhh0).reshape(4, H)
    bias1 = (bih1 + bhh1).reshape(4, H)

    num_p, den_p = _gat_edges_sc(xT, src, dst, csv, cdv, zrows)

    return _dense_chain(num_p, den_p, wg_big, bg_big, bias0, bias1,
                        W_lin, b_lin, wih0, whh0, wih1, whh1)


# speedup vs baseline: 1.3478x; 1.0008x over previous
import functools

import jax
import jax.numpy as jnp
from jax import lax
from jax.experimental import pallas as pl
from jax.experimental.pallas import tpu as pltpu
from jax.experimental.pallas import tpu_sc as plsc

N = 207
NP = 208
HID = 8
B = 8
S = 12
T = B * S
TP = 128
E = 3312
EP = 3328
NW = 32
EPW = EP // NW
H = N * HID
CH = 72


def _gat_edges_sc(xT, src, dst, csv, cdv, zrows):
    mesh = plsc.VectorSubcoreMesh(core_axis_name="c", subcore_axis_name="s")

    @functools.partial(
        pl.kernel,
        out_type=(
            jax.ShapeDtypeStruct((2, NP, TP), jnp.float32),
            jax.ShapeDtypeStruct((2, NP, TP), jnp.float32),
        ),
        mesh=mesh,
        scratch_types=[
            pltpu.VMEM((EPW,), jnp.int32),
            pltpu.VMEM((EPW,), jnp.int32),
            pltpu.VMEM((EPW, TP), jnp.float32),
            pltpu.VMEM((EPW, TP), jnp.float32),
            pltpu.VMEM((EPW, TP), jnp.float32),
            pltpu.VMEM((EPW, TP), jnp.float32),
            pltpu.VMEM((16,), jnp.float32),
            pltpu.VMEM((16,), jnp.float32),
            pltpu.VMEM_SHARED((NP, TP), jnp.float32),
            pltpu.VMEM_SHARED((NP, TP), jnp.float32),
            pltpu.SemaphoreType.DMA,
        ],
    )
    def k(xT_hbm, src_hbm, dst_hbm, cs_hbm, cd_hbm, z_hbm,
          num_hbm, den_hbm,
          src_v, dst_v, xs_v, xd_v, e_v, n_v, cs_v, cd_v,
          num_sh, den_sh, sem):
        c = lax.axis_index("c")
        sid = lax.axis_index("s")
        base = (c * 16 + sid) * EPW
        pltpu.sync_copy(src_hbm.at[pl.ds(base, EPW)], src_v)
        pltpu.sync_copy(dst_hbm.at[pl.ds(base, EPW)], dst_v)
        pltpu.sync_copy(cs_hbm, cs_v)
        pltpu.sync_copy(cd_hbm, cd_v)

        @pl.when(sid == 0)
        def _():
            pltpu.sync_copy(z_hbm, num_sh)
            pltpu.sync_copy(z_hbm, den_sh)

        pltpu.async_copy(xT_hbm.at[src_v], xs_v, sem).wait()
        pltpu.async_copy(xT_hbm.at[dst_v], xd_v, sem).wait()
        cs = cs_v[...]
        cd = cd_v[...]

        @pl.loop(0, EPW)
        def _(r):
            for j in range(TP // 16):
                sl = pl.ds(j * 16, 16)
                xs = xs_v[r, sl]
                xd = xd_v[r, sl]
                a = xs * cs + xd * cd
                a = jnp.where(a > 0.0, a, 0.2 * a)
                e = jnp.exp(a)
                e_v[r, sl] = e
                n_v[r, sl] = e * xs

        plsc.subcore_barrier()
        pltpu.sync_copy(n_v, num_sh.at[dst_v], add=True)
        pltpu.sync_copy(e_v, den_sh.at[dst_v], add=True)
        plsc.subcore_barrier()

        @pl.when(sid == 0)
        def _():
            pltpu.sync_copy(num_sh, num_hbm.at[c])
            pltpu.sync_copy(den_sh, den_hbm.at[c])

    return k(xT, src, dst, csv, cdv, zrows)


def _dotT(a, b):
    return lax.dot_general(a, b, (((1,), (1,)), ((), ())),
                           preferred_element_type=jnp.float32)


def _dotTb(a, b):
    return lax.dot_general(a.astype(jnp.bfloat16), b, (((1,), (1,)), ((), ())),
                           preferred_element_type=jnp.float32)


def _dense_body(num_ref, den_ref, wgt_ref, bg_ref, bias0_ref, bias1_ref,
                wl_ref, bl_ref, wih0_hbm, whh0_hbm, wih1_hbm, whh1_hbm,
                out_ref,
                fwin, wbf, u0, h1, sem):
    seq = ([(wih0_hbm, k) for k in range(4)] + [(whh0_hbm, k) for k in range(4)]
           + [(wih1_hbm, k) for k in range(4)] + [(whh1_hbm, k) for k in range(4)])

    def _cp(i):
        w_hbm, k = seq[i]
        return pltpu.make_async_copy(w_hbm.at[k], fwin.at[i % 2], sem.at[i % 2])

    def start(i):
        if i < len(seq):
            _cp(i).start()

    def cast_slab(i, k):
        @pl.loop(0, H // CH)
        def _(j):
            r = pl.ds(pl.multiple_of(j * CH, 8), CH)
            wbf[k, r, :] = fwin[i % 2, r, :].astype(jnp.bfloat16)

    def lstm_step(u, h, c, s):
        row = pl.multiple_of(8 * s, 8)
        gi = u[0, pl.ds(row, B), :] + _dotTb(h, wbf[0])
        gf = u[1, pl.ds(row, B), :] + _dotTb(h, wbf[1])
        gg = u[2, pl.ds(row, B), :] + _dotTb(h, wbf[2])
        go = u[3, pl.ds(row, B), :] + _dotTb(h, wbf[3])
        c = jax.nn.sigmoid(gf) * c + jax.nn.sigmoid(gi) * jnp.tanh(gg)
        h = jax.nn.sigmoid(go) * jnp.tanh(c)
        return h, c

    start(0)
    start(1)

    s_mat = (num_ref[0] + num_ref[1]) / (den_ref[0] + den_ref[1] + 1e-16)
    ii = lax.broadcasted_iota(jnp.int32, (T, TP), 0)
    jj = lax.broadcasted_iota(jnp.int32, (T, TP), 1)
    eye = jnp.where(ii == jj, 1.0, 0.0)
    sT = _dotT(eye, s_mat)
    x = jnp.maximum(_dotT(sT, wgt_ref[...]) + bg_ref[...][None, :], 0.0)

    xb = x.astype(jnp.bfloat16)
    for k in range(4):
        i = k
        _cp(i).wait()
        cast_slab(i, k)
        u0[k] = _dotTb(xb, wbf[k]) + bias0_ref[k][None, :]
        start(i + 2)

    for k in range(4):
        i = 4 + k
        _cp(i).wait()
        cast_slab(i, k)
        start(i + 2)

    def step0(s, carry):
        h, c = lstm_step(u0, *carry, s)
        h1[pl.ds(pl.multiple_of(8 * s, 8), B), :] = h
        return h, c

    z = jnp.zeros((B, H), jnp.float32)
    lax.fori_loop(0, S, step0, (z, z))

    h1b = h1[...].astype(jnp.bfloat16)
    for k in range(4):
        i = 8 + k
        _cp(i).wait()
        cast_slab(i, k)
        u0[k] = _dotTb(h1b, wbf[k]) + bias1_ref[k][None, :]
        start(i + 2)

    for k in range(4):
        i = 12 + k
        _cp(i).wait()
        cast_slab(i, k)
        start(i + 2)

    def step1(s, carry):
        return lstm_step(u0, *carry, s)

    h, _ = lax.fori_loop(0, S, step1, (z, z))
    out_ref[...] = _dotT(h, wl_ref[...]) + bl_ref[...][None, :]


def _dense_chain(num_p, den_p, wg_big, bg_big, bias0, bias1, wlin, blin,
                 wih0, whh0, wih1, whh1):
    vmem = pl.BlockSpec(memory_space=pltpu.VMEM)
    hbm = pl.BlockSpec(memory_space=pl.ANY)
    return pl.pallas_call(
        _dense_body,
        in_specs=[vmem] * 8 + [hbm] * 4,
        out_specs=vmem,
        out_shape=jax.ShapeDtypeStruct((B, N), jnp.float32),
        scratch_shapes=[
            pltpu.VMEM((2, H, H), jnp.float32),
            pltpu.VMEM((4, H, H), jnp.bfloat16),
            pltpu.VMEM((4, T, H), jnp.float32),
            pltpu.VMEM((T, H), jnp.float32),
            pltpu.SemaphoreType.DMA((2,)),
        ],
    )(num_p, den_p, wg_big, bg_big, bias0, bias1, wlin, blin,
      wih0, whh0, wih1, whh1)


def kernel(x_sequence, edge_index, W_gat, a_src, a_dst, b_gat,
           Wih0, Whh0, bih0, bhh0, Wih1, Whh1, bih1, bhh1, W_lin, b_lin):
    f32 = jnp.float32
    xT = jnp.transpose(x_sequence, (2, 1, 0)).reshape(N, T)
    xT = jnp.pad(xT, ((0, 1), (0, TP - T)))
    pad = jnp.full((EP - E,), N, jnp.int32)
    src = jnp.concatenate([edge_index[0], pad])
    dst = jnp.concatenate([edge_index[1], pad])
    wrow = W_gat[0]
    csv = jnp.full((16,), jnp.dot(wrow, a_src), f32)
    cdv = jnp.full((16,), jnp.dot(wrow, a_dst), f32)
    zrows = jnp.zeros((NP, TP), f32)
    wg_big = jnp.kron(jnp.eye(N, NP, dtype=f32), wrow[:, None])
    bg_big = jnp.tile(b_gat, N)
    wih0 = Wih0.reshape(4, H, H)
    whh0 = Whh0.reshape(4, H, H)
    wih1 = Wih1.reshape(4, H, H)
    whh1 = Whh1.reshape(4, H, H)
    bias0 = (bih0 + bhh0).reshape(4, H)
    bias1 = (bih1 + bhh1).reshape(4, H)

    num_p, den_p = _gat_edges_sc(xT, src, dst, csv, cdv, zrows)

    return _dense_chain(num_p, den_p, wg_big, bg_big, bias0, bias1,
                        W_lin, b_lin, wih0, whh0, wih1, whh1)


# speedup vs baseline: 1.4229x; 1.0557x over previous
import functools

import jax
import jax.numpy as jnp
from jax import lax
from jax.experimental import pallas as pl
from jax.experimental.pallas import tpu as pltpu
from jax.experimental.pallas import tpu_sc as plsc

N = 207
NP = 208
HID = 8
B = 8
S = 12
T = B * S
TP = 128
E = 3312
EP = 3328
NW = 32
EPW = EP // NW
H = N * HID
CH = 72


def _gat_edges_sc(xT, src, dst, csv, cdv, zrows):
    mesh = plsc.VectorSubcoreMesh(core_axis_name="c", subcore_axis_name="s")

    @functools.partial(
        pl.kernel,
        out_type=(
            jax.ShapeDtypeStruct((2, NP, TP), jnp.float32),
            jax.ShapeDtypeStruct((2, NP, TP), jnp.float32),
        ),
        mesh=mesh,
        scratch_types=[
            pltpu.VMEM((EPW,), jnp.int32),
            pltpu.VMEM((EPW,), jnp.int32),
            pltpu.VMEM((EPW, TP), jnp.float32),
            pltpu.VMEM((EPW, TP), jnp.float32),
            pltpu.VMEM((EPW, TP), jnp.float32),
            pltpu.VMEM((EPW, TP), jnp.float32),
            pltpu.VMEM((16,), jnp.float32),
            pltpu.VMEM((16,), jnp.float32),
            pltpu.VMEM_SHARED((NP, TP), jnp.float32),
            pltpu.VMEM_SHARED((NP, TP), jnp.float32),
            pltpu.SemaphoreType.DMA,
        ],
    )
    def k(xT_hbm, src_hbm, dst_hbm, cs_hbm, cd_hbm, z_hbm,
          num_hbm, den_hbm,
          src_v, dst_v, xs_v, xd_v, e_v, n_v, cs_v, cd_v,
          num_sh, den_sh, sem):
        c = lax.axis_index("c")
        sid = lax.axis_index("s")
        base = (c * 16 + sid) * EPW
        pltpu.sync_copy(src_hbm.at[pl.ds(base, EPW)], src_v)
        pltpu.sync_copy(dst_hbm.at[pl.ds(base, EPW)], dst_v)
        pltpu.sync_copy(cs_hbm, cs_v)
        pltpu.sync_copy(cd_hbm, cd_v)

        @pl.when(sid == 0)
        def _():
            pltpu.sync_copy(z_hbm, num_sh)
            pltpu.sync_copy(z_hbm, den_sh)

        pltpu.async_copy(xT_hbm.at[src_v], xs_v, sem).wait()
        pltpu.async_copy(xT_hbm.at[dst_v], xd_v, sem).wait()
        cs = cs_v[...]
        cd = cd_v[...]

        @pl.loop(0, EPW)
        def _(r):
            for j in range(T // 16):
                sl = pl.ds(j * 16, 16)
                xs = xs_v[r, sl]
                xd = xd_v[r, sl]
                a = xs * cs + xd * cd
                a = jnp.where(a > 0.0, a, 0.2 * a)
                e = jnp.exp(a)
                e_v[r, sl] = e
                n_v[r, sl] = e * xs

        plsc.subcore_barrier()
        pltpu.sync_copy(n_v, num_sh.at[dst_v], add=True)
        pltpu.sync_copy(e_v, den_sh.at[dst_v], add=True)
        plsc.subcore_barrier()

        @pl.when(sid == 0)
        def _():
            pltpu.sync_copy(num_sh, num_hbm.at[c])
            pltpu.sync_copy(den_sh, den_hbm.at[c])

    return k(xT, src, dst, csv, cdv, zrows)


def _dotT(a, b):
    return lax.dot_general(a, b, (((1,), (1,)), ((), ())),
                           preferred_element_type=jnp.float32)


def _dotTb(a, b):
    return lax.dot_general(a.astype(jnp.bfloat16), b, (((1,), (1,)), ((), ())),
                           preferred_element_type=jnp.float32)


def _dense_body(num_ref, den_ref, wgt_ref, bg_ref, bias0_ref, bias1_ref,
                wl_ref, bl_ref, wih0_hbm, whh0_hbm, wih1_hbm, whh1_hbm,
                out_ref,
                fwin, wbf, u0, h1, sem):
    seq = ([(wih0_hbm, k) for k in range(4)] + [(whh0_hbm, k) for k in range(4)]
           + [(wih1_hbm, k) for k in range(4)] + [(whh1_hbm, k) for k in range(4)])

    def _cp(i):
        w_hbm, k = seq[i]
        return pltpu.make_async_copy(w_hbm.at[k], fwin.at[i % 2], sem.at[i % 2])

    def start(i):
        if i < len(seq):
            _cp(i).start()

    def cast_slab(i, k):
        @pl.loop(0, H // CH)
        def _(j):
            r = pl.ds(pl.multiple_of(j * CH, 8), CH)
            wbf[k, r, :] = fwin[i % 2, r, :].astype(jnp.bfloat16)

    def lstm_step(u, h, c, s):
        row = pl.multiple_of(8 * s, 8)
        gi = u[0, pl.ds(row, B), :] + _dotTb(h, wbf[0])
        gf = u[1, pl.ds(row, B), :] + _dotTb(h, wbf[1])
        gg = u[2, pl.ds(row, B), :] + _dotTb(h, wbf[2])
        go = u[3, pl.ds(row, B), :] + _dotTb(h, wbf[3])
        c = jax.nn.sigmoid(gf) * c + jax.nn.sigmoid(gi) * jnp.tanh(gg)
        h = jax.nn.sigmoid(go) * jnp.tanh(c)
        return h, c

    def lstm_step0(u):
        c = jax.nn.sigmoid(u[0, pl.ds(0, B), :]) * jnp.tanh(u[2, pl.ds(0, B), :])
        h = jax.nn.sigmoid(u[3, pl.ds(0, B), :]) * jnp.tanh(c)
        return h, c

    start(0)
    start(1)

    s_mat = (num_ref[0] + num_ref[1]) / (den_ref[0] + den_ref[1] + 1e-16)
    s_mat = jnp.where(jnp.isfinite(s_mat), s_mat, 0.0)
    ii = lax.broadcasted_iota(jnp.int32, (T, TP), 0)
    jj = lax.broadcasted_iota(jnp.int32, (T, TP), 1)
    eye = jnp.where(ii == jj, 1.0, 0.0)
    sT = _dotT(eye, s_mat)
    x = jnp.maximum(_dotT(sT, wgt_ref[...]) + bg_ref[...][None, :], 0.0)

    xb = x.astype(jnp.bfloat16)
    for k in range(4):
        i = k
        _cp(i).wait()
        cast_slab(i, k)
        u0[k] = _dotTb(xb, wbf[k]) + bias0_ref[k][None, :]
        start(i + 2)

    for k in range(4):
        i = 4 + k
        _cp(i).wait()
        cast_slab(i, k)
        start(i + 2)

    def step0(s, carry):
        h, c = lstm_step(u0, *carry, s)
        h1[pl.ds(pl.multiple_of(8 * s, 8), B), :] = h
        return h, c

    hc = lstm_step0(u0)
    h1[pl.ds(0, B), :] = hc[0]
    lax.fori_loop(1, S, step0, hc)

    h1b = h1[...].astype(jnp.bfloat16)
    for k in range(4):
        i = 8 + k
        _cp(i).wait()
        cast_slab(i, k)
        u0[k] = _dotTb(h1b, wbf[k]) + bias1_ref[k][None, :]
        start(i + 2)

    for k in range(4):
        i = 12 + k
        _cp(i).wait()
        cast_slab(i, k)
        start(i + 2)

    def step1(s, carry):
        return lstm_step(u0, *carry, s)

    h, _ = lax.fori_loop(1, S, step1, lstm_step0(u0))
    out_ref[...] = _dotT(h, wl_ref[...]) + bl_ref[...][None, :]


def _dense_chain(num_p, den_p, wg_big, bg_big, bias0, bias1, wlin, blin,
                 wih0, whh0, wih1, whh1):
    vmem = pl.BlockSpec(memory_space=pltpu.VMEM)
    hbm = pl.BlockSpec(memory_space=pl.ANY)
    return pl.pallas_call(
        _dense_body,
        in_specs=[vmem] * 8 + [hbm] * 4,
        out_specs=vmem,
        out_shape=jax.ShapeDtypeStruct((B, N), jnp.float32),
        scratch_shapes=[
            pltpu.VMEM((2, H, H), jnp.float32),
            pltpu.VMEM((4, H, H), jnp.bfloat16),
            pltpu.VMEM((4, T, H), jnp.float32),
            pltpu.VMEM((T, H), jnp.float32),
            pltpu.SemaphoreType.DMA((2,)),
        ],
    )(num_p, den_p, wg_big, bg_big, bias0, bias1, wlin, blin,
      wih0, whh0, wih1, whh1)


def kernel(x_sequence, edge_index, W_gat, a_src, a_dst, b_gat,
           Wih0, Whh0, bih0, bhh0, Wih1, Whh1, bih1, bhh1, W_lin, b_lin):
    f32 = jnp.float32
    xT = jnp.transpose(x_sequence, (2, 1, 0)).reshape(N, T)
    xT = jnp.pad(xT, ((0, 1), (0, TP - T)))
    pad = jnp.full((EP - E,), N, jnp.int32)
    src = jnp.concatenate([edge_index[0], pad])
    dst = jnp.concatenate([edge_index[1], pad])
    wrow = W_gat[0]
    csv = jnp.full((16,), jnp.dot(wrow, a_src), f32)
    cdv = jnp.full((16,), jnp.dot(wrow, a_dst), f32)
    zrows = jnp.zeros((NP, TP), f32)
    wg_big = jnp.kron(jnp.eye(N, NP, dtype=f32), wrow[:, None])
    bg_big = jnp.tile(b_gat, N)
    wih0 = Wih0.reshape(4, H, H)
    whh0 = Whh0.reshape(4, H, H)
    wih1 = Wih1.reshape(4, H, H)
    whh1 = Whh1.reshape(4, H, H)
    bias0 = (bih0 + bhh0).reshape(4, H)
    bias1 = (bih1 + bhh1).reshape(4, H)

    num_p, den_p = _gat_edges_sc(xT, src, dst, csv, cdv, zrows)

    return _dense_chain(num_p, den_p, wg_big, bg_big, bias0, bias1,
                        W_lin, b_lin, wih0, whh0, wih1, whh1)
